# Initial kernel scaffold; baseline (speedup 1.0000x reference)
import jax
import jax.numpy as jnp
from jax.experimental import pallas as pl


def kernel(prev_state_h, prev_state_c, prev_relation, queries, actions_id, rel_emb, W_ih, W_hh, b_ih, b_hh, W1, b1, W2, b2):
    raise NotImplementedError("write your pallas kernel here")


# speedup vs baseline: 24.0730x; 24.0730x over previous
import functools

import jax
import jax.numpy as jnp
from jax import lax
from jax.experimental import pallas as pl
from jax.experimental.pallas import tpu as pltpu
from jax.experimental.pallas import tpu_sc as plsc

B, MO, NR, RE, SE, AE, HID = 4096, 200, 1000, 128, 128, 128, 256
NRP = 1024
MOP = 208
NEG = -99999.0
_BD = 512
_BF = 512
_NC, _NS = 2, 16
_NW = _NC * _NS
_RPT = B // _NW
_CH = 64
_HP = jax.lax.Precision.HIGHEST


def _dense_body(rel_ref, qry_ref, h_ref, c_ref, remb_ref, rembT_ref,
                wihT_ref, whhT_ref, bih_ref, bhh_ref, w1T_ref, b1_ref,
                w2T_ref, b2_ref, hout_ref, cout_ref, sall_ref):
    f32 = jnp.float32
    bf16 = jnp.bfloat16
    iota = lax.broadcasted_iota(jnp.int32, (_BD, NRP), 1)
    remb16 = remb_ref[...].astype(bf16)
    oh_prev = (rel_ref[...] == iota).astype(bf16)
    prev_action16 = jnp.dot(oh_prev, remb16,
                            preferred_element_type=f32).astype(bf16)
    gates = (jnp.dot(prev_action16, wihT_ref[...].astype(bf16),
                     preferred_element_type=f32)
             + bih_ref[...]
             + jnp.dot(h_ref[...].astype(bf16), whhT_ref[...].astype(bf16),
                       preferred_element_type=f32)
             + bhh_ref[...])
    i = jax.nn.sigmoid(gates[:, :SE])
    f = jax.nn.sigmoid(gates[:, SE:2 * SE])
    g = jnp.tanh(gates[:, 2 * SE:3 * SE])
    o = jax.nn.sigmoid(gates[:, 3 * SE:])
    c_new = f * c_ref[...] + i * g
    h_new = o * jnp.tanh(c_new)
    oh_q = (qry_ref[...] == iota).astype(bf16)
    qemb16 = jnp.dot(oh_q, remb16, preferred_element_type=f32).astype(bf16)
    sq16 = jnp.concatenate([h_new.astype(bf16), qemb16], axis=1)
    hidden = jnp.maximum(jnp.dot(sq16, w1T_ref[...].astype(bf16),
                                 preferred_element_type=f32) + b1_ref[...], 0.0)
    mlp = jnp.maximum(jnp.dot(hidden.astype(bf16), w2T_ref[...].astype(bf16),
                              preferred_element_type=f32) + b2_ref[...], 0.0)
    hout_ref[...] = h_new
    cout_ref[...] = c_new
    sall_ref[...] = jnp.dot(mlp, rembT_ref[...], precision=_HP,
                            preferred_element_type=f32)


def _dense_specs():
    blocked = lambda i: (i, 0)
    full = lambda i: (0, 0)
    in_specs = [
        pl.BlockSpec((_BD, 1), blocked),
        pl.BlockSpec((_BD, 1), blocked),
        pl.BlockSpec((_BD, SE), blocked),
        pl.BlockSpec((_BD, SE), blocked),
        pl.BlockSpec((NRP, RE), full),
        pl.BlockSpec((RE, NRP), full),
        pl.BlockSpec((AE, 4 * SE), full),
        pl.BlockSpec((SE, 4 * SE), full),
        pl.BlockSpec((1, 4 * SE), full),
        pl.BlockSpec((1, 4 * SE), full),
        pl.BlockSpec((SE + RE, HID), full),
        pl.BlockSpec((1, HID), full),
        pl.BlockSpec((HID, AE), full),
        pl.BlockSpec((1, AE), full),
    ]
    out_specs = [
        pl.BlockSpec((_BD, SE), blocked),
        pl.BlockSpec((_BD, SE), blocked),
        pl.BlockSpec((_BD, NRP), blocked),
    ]
    out_shape = [
        jax.ShapeDtypeStruct((B, SE), jnp.float32),
        jax.ShapeDtypeStruct((B, SE), jnp.float32),
        jax.ShapeDtypeStruct((B, NRP), jnp.float32),
    ]
    return dict(grid=(B // _BD,), in_specs=in_specs, out_specs=out_specs,
                out_shape=out_shape)


def _dense(*args):
    sp = _dense_specs()
    return pl.pallas_call(_dense_body, grid=sp["grid"], in_specs=sp["in_specs"],
                          out_specs=sp["out_specs"], out_shape=sp["out_shape"])(*args)


def _sc_gather(sall, aid_pad):
    mesh = plsc.VectorSubcoreMesh(core_axis_name="c", subcore_axis_name="s")

    @functools.partial(
        pl.kernel, mesh=mesh,
        compiler_params=pltpu.CompilerParams(needs_layout_passes=False),
        out_type=jax.ShapeDtypeStruct((B * MOP,), jnp.float32),
        scratch_types=[
            pltpu.VMEM((_CH * NRP,), jnp.float32),
            pltpu.VMEM((_CH * MOP,), jnp.int32),
            pltpu.VMEM((_CH * MOP,), jnp.float32),
        ],
    )
    def k(sall_hbm, aid_hbm, out_hbm, sc_v, idx_v, og_v):
        wid = lax.axis_index("s") * _NC + lax.axis_index("c")
        base = wid * _RPT

        def chunk(ci, carry):
            row0 = base + ci * _CH
            pltpu.sync_copy(sall_hbm.at[pl.ds(row0 * NRP, _CH * NRP)], sc_v)
            pltpu.sync_copy(aid_hbm.at[pl.ds(row0 * MOP, _CH * MOP)], idx_v)

            def row(r, c2):
                rbase = jnp.full((16,), r * NRP, jnp.int32)
                for j in range(MOP // 16):
                    idx = idx_v[pl.ds(r * MOP + j * 16, 16)] + rbase
                    og_v[pl.ds(r * MOP + j * 16, 16)] = plsc.load_gather(
                        sc_v, [idx])
                return c2

            lax.fori_loop(0, _CH, row, 0)
            pltpu.sync_copy(og_v, out_hbm.at[pl.ds(row0 * MOP, _CH * MOP)])
            return carry

        lax.fori_loop(0, _RPT // _CH, chunk, 0)

    return k(sall.reshape(B * NRP), aid_pad.reshape(B * MOP)).reshape(B, MOP)


def _fin_body(sg_ref, aid_ref, g_ref, loss_ref, logits_ref, act_ref, chosen_ref):
    sg = sg_ref[...]
    aid = aid_ref[...]
    masked = jnp.where(aid == 0, NEG, sg)
    y = masked + g_ref[...]
    lane = lax.broadcasted_iota(jnp.int32, (_BF, MOP), 1)
    ymax = jnp.max(y, axis=1, keepdims=True)
    amax = jnp.min(jnp.where(y == ymax, lane, MOP), axis=1, keepdims=True)
    valid = lane < MO
    mmax = jnp.max(jnp.where(valid, masked, -jnp.inf), axis=1, keepdims=True)
    sh = masked - mmax
    ex = jnp.where(valid, jnp.exp(sh), 0.0)
    lse = jnp.log(jnp.sum(ex, axis=1, keepdims=True))
    logits = sh - lse
    sel = lane == amax
    loss_ref[...] = -jnp.sum(jnp.where(sel, logits, 0.0), axis=1, keepdims=True)
    logits_ref[...] = logits[:, :MO]
    act_ref[...] = amax
    chosen_ref[...] = jnp.sum(jnp.where(sel, aid, 0), axis=1, keepdims=True)


def _fin_specs():
    blocked = lambda i: (i, 0)
    in_specs = [
        pl.BlockSpec((_BF, MOP), blocked),
        pl.BlockSpec((_BF, MOP), blocked),
        pl.BlockSpec((_BF, MOP), blocked),
    ]
    out_specs = [
        pl.BlockSpec((_BF, 1), blocked),
        pl.BlockSpec((_BF, MO), blocked),
        pl.BlockSpec((_BF, 1), blocked),
        pl.BlockSpec((_BF, 1), blocked),
    ]
    out_shape = [
        jax.ShapeDtypeStruct((B, 1), jnp.float32),
        jax.ShapeDtypeStruct((B, MO), jnp.float32),
        jax.ShapeDtypeStruct((B, 1), jnp.int32),
        jax.ShapeDtypeStruct((B, 1), jnp.int32),
    ]
    return dict(grid=(B // _BF,), in_specs=in_specs, out_specs=out_specs,
                out_shape=out_shape)


def _finalize(sg, aid_pad, gp):
    sp = _fin_specs()
    return pl.pallas_call(_fin_body, grid=sp["grid"], in_specs=sp["in_specs"],
                          out_specs=sp["out_specs"], out_shape=sp["out_shape"])(
                              sg, aid_pad, gp)


def kernel(prev_state_h, prev_state_c, prev_relation, queries, actions_id,
           rel_emb, W_ih, W_hh, b_ih, b_hh, W1, b1, W2, b2):
    aid = actions_id[:, :, 0].astype(jnp.int32)
    aid_pad = jnp.pad(aid, ((0, 0), (0, MOP - MO)))
    remb_pad = jnp.pad(rel_emb, ((0, NRP - NR), (0, 0)))
    g = jax.random.gumbel(jax.random.key(42), (B, MO), jnp.float32)
    gp = jnp.pad(g, ((0, 0), (0, MOP - MO)), constant_values=-1e30)
    rel2 = prev_relation.astype(jnp.int32).reshape(B, 1)
    qry2 = queries.astype(jnp.int32).reshape(B, 1)
    h_new, c_new, sall = _dense(
        rel2, qry2, prev_state_h, prev_state_c, remb_pad, remb_pad.T,
        W_ih.T, W_hh.T, b_ih.reshape(1, -1), b_hh.reshape(1, -1),
        W1.T, b1.reshape(1, -1), W2.T, b2.reshape(1, -1))
    sg = _sc_gather(sall, aid_pad)
    loss, logits, act, chosen = _finalize(sg, aid_pad, gp)
    return (loss.reshape(B), logits, act.reshape(B), chosen.reshape(B),
            h_new, c_new)
